# baseline (speedup 1.0000x reference)
import functools

import jax
import jax.numpy as jnp
from jax import lax
from jax.experimental import pallas as pl
from jax.experimental.pallas import tpu as pltpu
from jax.experimental.pallas import tpu_sc as plsc

VOCAB = 100000
EMB = 64
B = 4096
L = 200

N_TOK = B * L
CHUNK = 128
N_CHUNKS = N_TOK // CHUNK
NW = 32
CPW = N_CHUNKS // NW


def _body(x_hbm, emb_hbm, pos_hbm, out_hbm,
          idx0, idx1, rows0, rows1, pos_sh,
          sem_p0, sem_p1, sem_i0, sem_i1, sem_g, sem_s0, sem_s1):
    idx_v = (idx0, idx1)
    rows_v = (rows0, rows1)
    sem_p = (sem_p0, sem_p1)
    sem_i = (sem_i0, sem_i1)
    sem_s = (sem_s0, sem_s1)

    sid = lax.axis_index("s")
    wid = sid * 2 + lax.axis_index("c")
    base = wid * CPW
    end = base + CPW

    @pl.when(sid == 0)
    def _init():
        pltpu.sync_copy(pos_hbm.at[pl.ds(0, L)], pos_sh.at[pl.ds(0, L)])
        pltpu.sync_copy(pos_hbm.at[pl.ds(0, L)], pos_sh.at[pl.ds(L, L)])

    plsc.subcore_barrier()

    def start_load(c, b):
        p0 = lax.rem(c * CHUNK, L)
        pltpu.async_copy(pos_sh.at[pl.ds(p0, CHUNK)], rows_v[b], sem_p[b])
        pltpu.async_copy(x_hbm.at[pl.ds(c * CHUNK, CHUNK)], idx_v[b], sem_i[b])

    def wait_load(b):
        pltpu.make_async_copy(pos_sh.at[pl.ds(0, CHUNK)], rows_v[b], sem_p[b]).wait()
        pltpu.make_async_copy(x_hbm.at[pl.ds(0, CHUNK)], idx_v[b], sem_i[b]).wait()

    def wait_store(b):
        pltpu.make_async_copy(rows_v[b], out_hbm.at[pl.ds(0, CHUNK)], sem_s[b]).wait()

    start_load(base, 0)
    pltpu.async_copy(pos_sh.at[pl.ds(0, CHUNK)], rows_v[1], sem_s1)

    @pl.loop(0, CPW, step=2)
    def _chunk(t):
        for db in range(2):
            c = base + t + db
            b = db
            o = 1 - db

            wait_store(o)
            start_load(lax.min(c + 1, end - 1), o)

            wait_load(b)
            pltpu.async_copy(emb_hbm.at[idx_v[b]], rows_v[b], sem_g, add=True).wait()
            pltpu.async_copy(rows_v[b], out_hbm.at[pl.ds(c * CHUNK, CHUNK)], sem_s[b])

    wait_store(1)
    wait_load(0)


@jax.jit
def kernel(x, emb_table, pos_table):
    x_flat = jnp.reshape(x, (N_TOK,))
    mesh = plsc.VectorSubcoreMesh(core_axis_name="c", subcore_axis_name="s")
    out = pl.kernel(
        _body,
        out_type=jax.ShapeDtypeStruct((N_TOK, EMB), jnp.float32),
        mesh=mesh,
        compiler_params=pltpu.CompilerParams(use_tc_tiling_on_sc=False),
        scratch_types=[
            pltpu.VMEM((CHUNK,), jnp.int32),
            pltpu.VMEM((CHUNK,), jnp.int32),
            pltpu.VMEM((CHUNK, EMB), jnp.float32),
            pltpu.VMEM((CHUNK, EMB), jnp.float32),
            pltpu.VMEM_SHARED((2 * L, EMB), jnp.float32),
            pltpu.SemaphoreType.DMA,
            pltpu.SemaphoreType.DMA,
            pltpu.SemaphoreType.DMA,
            pltpu.SemaphoreType.DMA,
            pltpu.SemaphoreType.DMA,
            pltpu.SemaphoreType.DMA,
            pltpu.SemaphoreType.DMA,
        ],
    )(x_flat, emb_table, pos_table)
    return jnp.reshape(out, (B, L, EMB))


# speedup vs baseline: 1.0693x; 1.0693x over previous
import functools

import jax
import jax.numpy as jnp
from jax import lax
from jax.experimental import pallas as pl
from jax.experimental.pallas import tpu as pltpu
from jax.experimental.pallas import tpu_sc as plsc

VOCAB = 100000
EMB = 64
B = 4096
L = 200

NW = 32
SPW = B // NW


def _body(x_hbm, emb_hbm, pos_hbm, out_hbm,
          idx0, idx1, rows0, rows1, pos_sh,
          sem_p0, sem_p1, sem_i0, sem_i1, sem_g, sem_s0, sem_s1):
    idx_v = (idx0, idx1)
    rows_v = (rows0, rows1)
    sem_p = (sem_p0, sem_p1)
    sem_i = (sem_i0, sem_i1)
    sem_s = (sem_s0, sem_s1)

    sid = lax.axis_index("s")
    wid = sid * 2 + lax.axis_index("c")
    base = wid * SPW
    end = base + SPW

    @pl.when(sid == 0)
    def _init():
        pltpu.sync_copy(pos_hbm.at[pl.ds(0, L)], pos_sh)

    plsc.subcore_barrier()

    def start_load(s, b):
        pltpu.async_copy(pos_sh, rows_v[b], sem_p[b])
        pltpu.async_copy(x_hbm.at[s], idx_v[b], sem_i[b])

    def wait_load(b):
        pltpu.make_async_copy(pos_sh, rows_v[b], sem_p[b]).wait()
        pltpu.make_async_copy(x_hbm.at[0], idx_v[b], sem_i[b]).wait()

    def wait_store(b):
        pltpu.make_async_copy(rows_v[b], out_hbm.at[0], sem_s[b]).wait()

    start_load(base, 0)
    pltpu.async_copy(pos_sh, rows_v[1], sem_s1)

    @pl.loop(0, SPW, step=2)
    def _seq(t):
        for db in range(2):
            s = base + t + db
            b = db
            o = 1 - db

            wait_store(o)
            start_load(lax.min(s + 1, end - 1), o)

            wait_load(b)
            pltpu.async_copy(emb_hbm.at[idx_v[b]], rows_v[b], sem_g, add=True).wait()
            pltpu.async_copy(rows_v[b], out_hbm.at[s], sem_s[b])

    wait_store(1)
    wait_load(0)


@jax.jit
def kernel(x, emb_table, pos_table):
    mesh = plsc.VectorSubcoreMesh(core_axis_name="c", subcore_axis_name="s")
    return pl.kernel(
        _body,
        out_type=jax.ShapeDtypeStruct((B, L, EMB), jnp.float32),
        mesh=mesh,
        compiler_params=pltpu.CompilerParams(use_tc_tiling_on_sc=False),
        scratch_types=[
            pltpu.VMEM((L,), jnp.int32),
            pltpu.VMEM((L,), jnp.int32),
            pltpu.VMEM((L, EMB), jnp.float32),
            pltpu.VMEM((L, EMB), jnp.float32),
            pltpu.VMEM_SHARED((L, EMB), jnp.float32),
            pltpu.SemaphoreType.DMA,
            pltpu.SemaphoreType.DMA,
            pltpu.SemaphoreType.DMA,
            pltpu.SemaphoreType.DMA,
            pltpu.SemaphoreType.DMA,
            pltpu.SemaphoreType.DMA,
            pltpu.SemaphoreType.DMA,
        ],
    )(x, emb_table, pos_table)
